# Initial kernel scaffold; baseline (speedup 1.0000x reference)
import jax
import jax.numpy as jnp
from jax.experimental import pallas as pl


def kernel(spatial_features, gt_boxes):
    raise NotImplementedError("write your pallas kernel here")


# speedup vs baseline: 9.2094x; 9.2094x over previous
import functools

import jax
import jax.numpy as jnp
from jax.experimental import pallas as pl
from jax.experimental.pallas import tpu as pltpu

POOL = 7
NLEV = 6


def _build_kernel(f_ref, st_ref, scratch, *, cb, h, w):
    k = pl.program_id(2)
    for kk in range(NLEV):
        @pl.when(k == kk)
        def _(kk=kk):
            if kk == 0:
                cur = f_ref[0]
            else:
                s = 1 << (kk - 1)
                prev = scratch[...]
                shifted = jnp.concatenate(
                    [prev[s:], jnp.broadcast_to(prev[h - 1:], (s, cb, w))],
                    axis=0)
                cur = jnp.maximum(prev, shifted)
            scratch[...] = cur
            st_ref[0, 0] = cur


def _pool_kernel(lvl_ref, rowa_ref, rowb_ref, ws_ref, we_ref, valid_ref,
                 *refs, c, w):
    row_refs = refs[:2 * POOL]
    o_ref = refs[2 * POOL]
    b = pl.program_id(0)
    n = pl.program_id(1)
    wids = jax.lax.broadcasted_iota(jnp.int32, (c, w), 1)
    neg = jnp.float32(-jnp.inf)
    masks = []
    for j in range(POOL):
        wsj = ws_ref[b, n, j]
        wej = we_ref[b, n, j]
        masks.append((wids >= wsj) & (wids < wej))
    rows = []
    for i in range(POOL):
        ra = row_refs[2 * i][0, 0, 0]
        rb = row_refs[2 * i + 1][0, 0, 0]
        row2 = jnp.maximum(ra, rb)
        cols = [jnp.max(jnp.where(masks[j], row2, neg), axis=1)
                for j in range(POOL)]
        rows.append(jnp.stack(cols, axis=1))
    vals = jnp.stack(rows, axis=1)
    vals = jnp.where(valid_ref[b, n] != 0, vals, jnp.float32(0.0))
    o_ref[0, 0] = vals


def _box_tables(gt_boxes, hdim, wdim):
    boxes = gt_boxes.astype(jnp.int32)
    valid = jnp.any(boxes != 0, axis=2).astype(jnp.int32)
    x1 = jnp.clip(boxes[..., 0], 0, wdim - 1)
    y1 = jnp.clip(boxes[..., 1], 0, hdim - 1)
    x2 = jnp.clip(boxes[..., 2], 0, wdim - 1)
    y2 = jnp.clip(boxes[..., 3], 0, hdim - 1)
    x1 = jnp.minimum(x1, x2)
    x2 = jnp.maximum(x1, x2)
    y1 = jnp.minimum(y1, y2)
    y2 = jnp.maximum(y1, y2)
    x2 = jnp.where(x1 == x2, x1 + 1, x2)
    y2 = jnp.where(y1 == y2, y1 + 1, y2)
    hr = y2 - y1
    wr = x2 - x1
    idx = jnp.arange(POOL)
    hs = y1[..., None] + (idx * hr[..., None]) // POOL
    he = y1[..., None] + ((idx + 1) * hr[..., None] + POOL - 1) // POOL
    ws = x1[..., None] + (idx * wr[..., None]) // POOL
    we = x1[..., None] + ((idx + 1) * wr[..., None] + POOL - 1) // POOL
    nh = he - hs
    lvl = 31 - jax.lax.clz(nh)
    rowa = hs
    rowb = he - jnp.left_shift(1, lvl)
    return lvl, rowa, rowb, ws, we, valid


def kernel(spatial_features, gt_boxes):
    bdim, cdim, hdim, wdim = spatial_features.shape
    ndim = gt_boxes.shape[1]

    ftr = jnp.transpose(spatial_features, (0, 2, 1, 3))
    cb = 8
    st = pl.pallas_call(
        functools.partial(_build_kernel, cb=cb, h=hdim, w=wdim),
        grid=(bdim, cdim // cb, NLEV),
        in_specs=[pl.BlockSpec((1, hdim, cb, wdim),
                               lambda b, ci, k: (b, 0, ci, 0))],
        out_specs=pl.BlockSpec((1, 1, hdim, cb, wdim),
                               lambda b, ci, k: (k, b, 0, ci, 0)),
        out_shape=jax.ShapeDtypeStruct((NLEV, bdim, hdim, cdim, wdim),
                                       jnp.float32),
        scratch_shapes=[pltpu.VMEM((hdim, cb, wdim), jnp.float32)],
        compiler_params=pltpu.CompilerParams(
            dimension_semantics=("arbitrary", "arbitrary", "arbitrary"),
        ),
    )(ftr)

    lvl, rowa, rowb, ws, we, valid = _box_tables(gt_boxes, hdim, wdim)

    def mk_index(i, which):
        def idx(b, n, lvl_r, rowa_r, rowb_r, ws_r, we_r, valid_r):
            row_r = rowa_r if which == 0 else rowb_r
            return (lvl_r[b, n, i], b, row_r[b, n, i], 0, 0)
        return idx

    in_specs = []
    for i in range(POOL):
        for which in (0, 1):
            in_specs.append(pl.BlockSpec((1, 1, 1, cdim, wdim),
                                         mk_index(i, which)))

    grid_spec = pltpu.PrefetchScalarGridSpec(
        num_scalar_prefetch=6,
        grid=(bdim, ndim),
        in_specs=in_specs,
        out_specs=pl.BlockSpec((1, 1, cdim, POOL, POOL),
                               lambda b, n, *_: (b, n, 0, 0, 0)),
    )
    out = pl.pallas_call(
        functools.partial(_pool_kernel, c=cdim, w=wdim),
        grid_spec=grid_spec,
        out_shape=jax.ShapeDtypeStruct((bdim, ndim, cdim, POOL, POOL),
                                       jnp.float32),
        compiler_params=pltpu.CompilerParams(
            dimension_semantics=("arbitrary", "arbitrary"),
        ),
    )(lvl, rowa, rowb, ws, we, valid, *([st] * (2 * POOL)))
    return out


# speedup vs baseline: 11.2420x; 1.2207x over previous
import functools

import jax
import jax.numpy as jnp
from jax.experimental import pallas as pl
from jax.experimental.pallas import tpu as pltpu

POOL = 7
NLEV = 6


def _build_kernel(f_ref, st_ref, scratch, *, cb, h, w):
    k = pl.program_id(2)
    for kk in range(NLEV):
        @pl.when(k == kk)
        def _(kk=kk):
            if kk == 0:
                cur = f_ref[0].astype(jnp.bfloat16)
            else:
                s = 1 << (kk - 1)
                prev = scratch[...]
                shifted = jnp.concatenate(
                    [prev[s:], jnp.broadcast_to(prev[h - 1:], (s, cb, w))],
                    axis=0)
                cur = jnp.maximum(prev, shifted)
            scratch[...] = cur
            st_ref[0, 0] = cur


def _pool_kernel(lvl_ref, rowa_ref, rowb_ref, ws_ref, we_ref, valid_ref,
                 *refs, c, w):
    row_refs = refs[:2 * POOL]
    o_ref = refs[2 * POOL]
    b = pl.program_id(0)
    n = pl.program_id(1)
    wids = jax.lax.broadcasted_iota(jnp.int32, (c, w), 1)
    neg = jnp.bfloat16(-jnp.inf)
    masks = []
    for j in range(POOL):
        wsj = ws_ref[b, n, j]
        wej = we_ref[b, n, j]
        masks.append((wids >= wsj) & (wids < wej))
    rows = []
    for i in range(POOL):
        ra = row_refs[2 * i][0, 0, 0]
        rb = row_refs[2 * i + 1][0, 0, 0]
        row2 = jnp.maximum(ra, rb)
        cols = [jnp.max(jnp.where(masks[j], row2, neg), axis=1)
                for j in range(POOL)]
        rows.append(jnp.stack(cols, axis=1))
    vals = jnp.stack(rows, axis=1).astype(jnp.float32)
    vals = jnp.where(valid_ref[b, n] != 0, vals, jnp.float32(0.0))
    o_ref[0, 0] = vals


def _box_tables(gt_boxes, hdim, wdim):
    boxes = gt_boxes.astype(jnp.int32)
    valid = jnp.any(boxes != 0, axis=2).astype(jnp.int32)
    x1 = jnp.clip(boxes[..., 0], 0, wdim - 1)
    y1 = jnp.clip(boxes[..., 1], 0, hdim - 1)
    x2 = jnp.clip(boxes[..., 2], 0, wdim - 1)
    y2 = jnp.clip(boxes[..., 3], 0, hdim - 1)
    x1 = jnp.minimum(x1, x2)
    x2 = jnp.maximum(x1, x2)
    y1 = jnp.minimum(y1, y2)
    y2 = jnp.maximum(y1, y2)
    x2 = jnp.where(x1 == x2, x1 + 1, x2)
    y2 = jnp.where(y1 == y2, y1 + 1, y2)
    hr = y2 - y1
    wr = x2 - x1
    idx = jnp.arange(POOL)
    hs = y1[..., None] + (idx * hr[..., None]) // POOL
    he = y1[..., None] + ((idx + 1) * hr[..., None] + POOL - 1) // POOL
    ws = x1[..., None] + (idx * wr[..., None]) // POOL
    we = x1[..., None] + ((idx + 1) * wr[..., None] + POOL - 1) // POOL
    nh = he - hs
    lvl = 31 - jax.lax.clz(nh)
    rowa = hs
    rowb = he - jnp.left_shift(1, lvl)
    return lvl, rowa, rowb, ws, we, valid


def kernel(spatial_features, gt_boxes):
    bdim, cdim, hdim, wdim = spatial_features.shape
    ndim = gt_boxes.shape[1]

    ftr = jnp.transpose(spatial_features, (0, 2, 1, 3))
    cb = 8
    st = pl.pallas_call(
        functools.partial(_build_kernel, cb=cb, h=hdim, w=wdim),
        grid=(bdim, cdim // cb, NLEV),
        in_specs=[pl.BlockSpec((1, hdim, cb, wdim),
                               lambda b, ci, k: (b, 0, ci, 0))],
        out_specs=pl.BlockSpec((1, 1, hdim, cb, wdim),
                               lambda b, ci, k: (k, b, 0, ci, 0)),
        out_shape=jax.ShapeDtypeStruct((NLEV, bdim, hdim, cdim, wdim),
                                       jnp.bfloat16),
        scratch_shapes=[pltpu.VMEM((hdim, cb, wdim), jnp.bfloat16)],
        compiler_params=pltpu.CompilerParams(
            dimension_semantics=("arbitrary", "arbitrary", "arbitrary"),
        ),
    )(ftr)

    lvl, rowa, rowb, ws, we, valid = _box_tables(gt_boxes, hdim, wdim)

    def mk_index(i, which):
        def idx(b, n, lvl_r, rowa_r, rowb_r, ws_r, we_r, valid_r):
            row_r = rowa_r if which == 0 else rowb_r
            return (lvl_r[b, n, i], b, row_r[b, n, i], 0, 0)
        return idx

    in_specs = []
    for i in range(POOL):
        for which in (0, 1):
            in_specs.append(pl.BlockSpec((1, 1, 1, cdim, wdim),
                                         mk_index(i, which)))

    grid_spec = pltpu.PrefetchScalarGridSpec(
        num_scalar_prefetch=6,
        grid=(bdim, ndim),
        in_specs=in_specs,
        out_specs=pl.BlockSpec((1, 1, cdim, POOL, POOL),
                               lambda b, n, *_: (b, n, 0, 0, 0)),
    )
    out = pl.pallas_call(
        functools.partial(_pool_kernel, c=cdim, w=wdim),
        grid_spec=grid_spec,
        out_shape=jax.ShapeDtypeStruct((bdim, ndim, cdim, POOL, POOL),
                                       jnp.float32),
        compiler_params=pltpu.CompilerParams(
            dimension_semantics=("arbitrary", "arbitrary"),
        ),
    )(lvl, rowa, rowb, ws, we, valid, *([st] * (2 * POOL)))
    return out


# speedup vs baseline: 12.9273x; 1.1499x over previous
import functools

import jax
import jax.numpy as jnp
from jax.experimental import pallas as pl
from jax.experimental.pallas import tpu as pltpu

POOL = 7
NLEV = 6


def _build_kernel(f_ref, st_ref, scratch, *, cb, h, w):
    k = pl.program_id(2)
    for kk in range(NLEV):
        @pl.when(k == kk)
        def _(kk=kk):
            if kk == 0:
                cur = jnp.swapaxes(f_ref[0].astype(jnp.bfloat16), 0, 1)
            else:
                s = 1 << (kk - 1)
                prev = scratch[...]
                shifted = jnp.concatenate(
                    [prev[s:], jnp.broadcast_to(prev[h - 1:], (s, cb, w))],
                    axis=0)
                cur = jnp.maximum(prev, shifted)
            scratch[...] = cur
            st_ref[0, 0] = cur


def _pool_kernel(lvl_ref, rowa_ref, rowb_ref, ws_ref, we_ref, valid_ref,
                 *refs, c, w):
    row_refs = refs[:2 * POOL]
    o_ref = refs[2 * POOL]
    b = pl.program_id(0)
    n = pl.program_id(1)
    wids = jax.lax.broadcasted_iota(jnp.int32, (c, w), 1)
    neg = jnp.bfloat16(-jnp.inf)
    masks = []
    for j in range(POOL):
        wsj = ws_ref[b, n, j]
        wej = we_ref[b, n, j]
        masks.append((wids >= wsj) & (wids < wej))
    rows = []
    for i in range(POOL):
        ra = row_refs[2 * i][0, 0, 0]
        rb = row_refs[2 * i + 1][0, 0, 0]
        row2 = jnp.maximum(ra, rb)
        cols = [jnp.max(jnp.where(masks[j], row2, neg), axis=1)
                for j in range(POOL)]
        rows.append(jnp.stack(cols, axis=1))
    vals = jnp.stack(rows, axis=1).astype(jnp.float32)
    vals = jnp.where(valid_ref[b, n] != 0, vals, jnp.float32(0.0))
    o_ref[0, 0] = vals


def _box_tables(gt_boxes, hdim, wdim):
    boxes = gt_boxes.astype(jnp.int32)
    valid = jnp.any(boxes != 0, axis=2).astype(jnp.int32)
    x1 = jnp.clip(boxes[..., 0], 0, wdim - 1)
    y1 = jnp.clip(boxes[..., 1], 0, hdim - 1)
    x2 = jnp.clip(boxes[..., 2], 0, wdim - 1)
    y2 = jnp.clip(boxes[..., 3], 0, hdim - 1)
    x1 = jnp.minimum(x1, x2)
    x2 = jnp.maximum(x1, x2)
    y1 = jnp.minimum(y1, y2)
    y2 = jnp.maximum(y1, y2)
    x2 = jnp.where(x1 == x2, x1 + 1, x2)
    y2 = jnp.where(y1 == y2, y1 + 1, y2)
    hr = y2 - y1
    wr = x2 - x1
    idx = jnp.arange(POOL)
    hs = y1[..., None] + (idx * hr[..., None]) // POOL
    he = y1[..., None] + ((idx + 1) * hr[..., None] + POOL - 1) // POOL
    ws = x1[..., None] + (idx * wr[..., None]) // POOL
    we = x1[..., None] + ((idx + 1) * wr[..., None] + POOL - 1) // POOL
    nh = he - hs
    lvl = 31 - jax.lax.clz(nh)
    rowa = hs
    rowb = he - jnp.left_shift(1, lvl)
    return lvl, rowa, rowb, ws, we, valid


def kernel(spatial_features, gt_boxes):
    bdim, cdim, hdim, wdim = spatial_features.shape
    ndim = gt_boxes.shape[1]

    cb = 8
    st = pl.pallas_call(
        functools.partial(_build_kernel, cb=cb, h=hdim, w=wdim),
        grid=(bdim, cdim // cb, NLEV),
        in_specs=[pl.BlockSpec((1, cb, hdim, wdim),
                               lambda b, ci, k: (b, ci, 0, 0))],
        out_specs=pl.BlockSpec((1, 1, hdim, cb, wdim),
                               lambda b, ci, k: (k, b, 0, ci, 0)),
        out_shape=jax.ShapeDtypeStruct((NLEV, bdim, hdim, cdim, wdim),
                                       jnp.bfloat16),
        scratch_shapes=[pltpu.VMEM((hdim, cb, wdim), jnp.bfloat16)],
        compiler_params=pltpu.CompilerParams(
            dimension_semantics=("arbitrary", "arbitrary", "arbitrary"),
        ),
    )(spatial_features)

    lvl, rowa, rowb, ws, we, valid = _box_tables(gt_boxes, hdim, wdim)

    def mk_index(i, which):
        def idx(b, n, lvl_r, rowa_r, rowb_r, ws_r, we_r, valid_r):
            row_r = rowa_r if which == 0 else rowb_r
            return (lvl_r[b, n, i], b, row_r[b, n, i], 0, 0)
        return idx

    in_specs = []
    for i in range(POOL):
        for which in (0, 1):
            in_specs.append(pl.BlockSpec((1, 1, 1, cdim, wdim),
                                         mk_index(i, which)))

    grid_spec = pltpu.PrefetchScalarGridSpec(
        num_scalar_prefetch=6,
        grid=(bdim, ndim),
        in_specs=in_specs,
        out_specs=pl.BlockSpec((1, 1, cdim, POOL, POOL),
                               lambda b, n, *_: (b, n, 0, 0, 0)),
    )
    out = pl.pallas_call(
        functools.partial(_pool_kernel, c=cdim, w=wdim),
        grid_spec=grid_spec,
        out_shape=jax.ShapeDtypeStruct((bdim, ndim, cdim, POOL, POOL),
                                       jnp.float32),
        compiler_params=pltpu.CompilerParams(
            dimension_semantics=("arbitrary", "arbitrary"),
        ),
    )(lvl, rowa, rowb, ws, we, valid, *([st] * (2 * POOL)))
    return out


# speedup vs baseline: 14.6897x; 1.1363x over previous
import functools

import jax
import jax.numpy as jnp
from jax.experimental import pallas as pl
from jax.experimental.pallas import tpu as pltpu

POOL = 7
NLEV = 6


def _build_kernel(f_ref, st_ref, *, cb, h, w):
    cur = jnp.swapaxes(f_ref[0].astype(jnp.bfloat16), 0, 1)
    st_ref[0, 0] = cur
    for k in range(1, NLEV):
        s = 1 << (k - 1)
        shifted = jnp.concatenate(
            [cur[s:], jnp.broadcast_to(cur[h - 1:], (s, cb, w))],
            axis=0)
        cur = jnp.maximum(cur, shifted)
        st_ref[k, 0] = cur


def _pool_kernel(lvl_ref, rowa_ref, rowb_ref, ws_ref, we_ref, valid_ref,
                 *refs, c, w):
    row_refs = refs[:2 * POOL]
    o_ref = refs[2 * POOL]
    b = pl.program_id(0)
    n = pl.program_id(1)
    wids = jax.lax.broadcasted_iota(jnp.int32, (1, w), 1)
    neg = jnp.bfloat16(-jnp.inf)
    masks = []
    for j in range(POOL):
        wsj = ws_ref[b, n, j]
        wej = we_ref[b, n, j]
        masks.append((wids >= wsj) & (wids < wej))
    rows = []
    for i in range(POOL):
        ra = row_refs[2 * i][0, 0, 0]
        rb = row_refs[2 * i + 1][0, 0, 0]
        row2 = jnp.maximum(ra, rb)
        cols = [jnp.max(jnp.where(masks[j], row2, neg), axis=1)
                for j in range(POOL)]
        rows.append(jnp.stack(cols, axis=1))
    vals = jnp.stack(rows, axis=1).astype(jnp.float32)
    vals = jnp.where(valid_ref[b, n] != 0, vals, jnp.float32(0.0))
    o_ref[0, 0] = vals


def _box_tables(gt_boxes, hdim, wdim):
    boxes = gt_boxes.astype(jnp.int32)
    valid = jnp.any(boxes != 0, axis=2).astype(jnp.int32)
    x1 = jnp.clip(boxes[..., 0], 0, wdim - 1)
    y1 = jnp.clip(boxes[..., 1], 0, hdim - 1)
    x2 = jnp.clip(boxes[..., 2], 0, wdim - 1)
    y2 = jnp.clip(boxes[..., 3], 0, hdim - 1)
    x1 = jnp.minimum(x1, x2)
    x2 = jnp.maximum(x1, x2)
    y1 = jnp.minimum(y1, y2)
    y2 = jnp.maximum(y1, y2)
    x2 = jnp.where(x1 == x2, x1 + 1, x2)
    y2 = jnp.where(y1 == y2, y1 + 1, y2)
    hr = y2 - y1
    wr = x2 - x1
    idx = jnp.arange(POOL)
    hs = y1[..., None] + (idx * hr[..., None]) // POOL
    he = y1[..., None] + ((idx + 1) * hr[..., None] + POOL - 1) // POOL
    ws = x1[..., None] + (idx * wr[..., None]) // POOL
    we = x1[..., None] + ((idx + 1) * wr[..., None] + POOL - 1) // POOL
    nh = he - hs
    lvl = 31 - jax.lax.clz(nh)
    rowa = hs
    rowb = he - jnp.left_shift(1, lvl)
    return lvl, rowa, rowb, ws, we, valid


def kernel(spatial_features, gt_boxes):
    bdim, cdim, hdim, wdim = spatial_features.shape
    ndim = gt_boxes.shape[1]

    cb = 8
    st = pl.pallas_call(
        functools.partial(_build_kernel, cb=cb, h=hdim, w=wdim),
        grid=(bdim, cdim // cb),
        in_specs=[pl.BlockSpec((1, cb, hdim, wdim),
                               lambda b, ci: (b, ci, 0, 0))],
        out_specs=pl.BlockSpec((NLEV, 1, hdim, cb, wdim),
                               lambda b, ci: (0, b, 0, ci, 0)),
        out_shape=jax.ShapeDtypeStruct((NLEV, bdim, hdim, cdim, wdim),
                                       jnp.bfloat16),
        compiler_params=pltpu.CompilerParams(
            dimension_semantics=("arbitrary", "arbitrary"),
            vmem_limit_bytes=100 * 1024 * 1024,
        ),
    )(spatial_features)

    lvl, rowa, rowb, ws, we, valid = _box_tables(gt_boxes, hdim, wdim)

    def mk_index(i, which):
        def idx(b, n, lvl_r, rowa_r, rowb_r, ws_r, we_r, valid_r):
            row_r = rowa_r if which == 0 else rowb_r
            return (lvl_r[b, n, i], b, row_r[b, n, i], 0, 0)
        return idx

    in_specs = []
    for i in range(POOL):
        for which in (0, 1):
            in_specs.append(pl.BlockSpec((1, 1, 1, cdim, wdim),
                                         mk_index(i, which)))

    grid_spec = pltpu.PrefetchScalarGridSpec(
        num_scalar_prefetch=6,
        grid=(bdim, ndim),
        in_specs=in_specs,
        out_specs=pl.BlockSpec((1, 1, cdim, POOL, POOL),
                               lambda b, n, *_: (b, n, 0, 0, 0)),
    )
    out = pl.pallas_call(
        functools.partial(_pool_kernel, c=cdim, w=wdim),
        grid_spec=grid_spec,
        out_shape=jax.ShapeDtypeStruct((bdim, ndim, cdim, POOL, POOL),
                                       jnp.float32),
        compiler_params=pltpu.CompilerParams(
            dimension_semantics=("arbitrary", "arbitrary"),
        ),
    )(lvl, rowa, rowb, ws, we, valid, *([st] * (2 * POOL)))
    return out


# speedup vs baseline: 15.6311x; 1.0641x over previous
import functools

import jax
import jax.numpy as jnp
from jax.experimental import pallas as pl
from jax.experimental.pallas import tpu as pltpu

POOL = 7
NLEV = 6


def _build_kernel(f_ref, st_ref, *, cb, h, w):
    cur = jnp.swapaxes(f_ref[0].astype(jnp.bfloat16), 0, 1)
    st_ref[0, 0] = cur
    for k in range(1, NLEV):
        s = 1 << (k - 1)
        shifted = jnp.concatenate(
            [cur[s:], jnp.broadcast_to(cur[h - 1:], (s, cb, w))],
            axis=0)
        cur = jnp.maximum(cur, shifted)
        st_ref[k, 0] = cur


def _pool_kernel(lvl_ref, rowa_ref, rowb_ref, ws_ref, we_ref, valid_ref,
                 *refs, c, w):
    row_refs = refs[:2 * POOL]
    o_ref = refs[2 * POOL]
    b = pl.program_id(0)
    n = pl.program_id(1)
    wids = jax.lax.broadcasted_iota(jnp.int32, (1, w), 1)
    neg = jnp.bfloat16(-jnp.inf)
    masks = []
    for j in range(POOL):
        wsj = ws_ref[b, n, j]
        wej = we_ref[b, n, j]
        masks.append((wids >= wsj) & (wids < wej))
    vmask = valid_ref[b, n] != 0
    for i in range(POOL):
        ra = row_refs[2 * i][0, 0, 0]
        rb = row_refs[2 * i + 1][0, 0, 0]
        row2 = jnp.maximum(ra, rb)
        cols = [jnp.max(jnp.where(masks[j], row2, neg), axis=1)
                for j in range(POOL)]
        vals = jnp.stack(cols, axis=1).astype(jnp.float32)
        vals = jnp.where(vmask, vals, jnp.float32(0.0))
        o_ref[0, 0, i] = vals


def _box_tables(gt_boxes, hdim, wdim):
    boxes = gt_boxes.astype(jnp.int32)
    valid = jnp.any(boxes != 0, axis=2).astype(jnp.int32)
    x1 = jnp.clip(boxes[..., 0], 0, wdim - 1)
    y1 = jnp.clip(boxes[..., 1], 0, hdim - 1)
    x2 = jnp.clip(boxes[..., 2], 0, wdim - 1)
    y2 = jnp.clip(boxes[..., 3], 0, hdim - 1)
    x1 = jnp.minimum(x1, x2)
    x2 = jnp.maximum(x1, x2)
    y1 = jnp.minimum(y1, y2)
    y2 = jnp.maximum(y1, y2)
    x2 = jnp.where(x1 == x2, x1 + 1, x2)
    y2 = jnp.where(y1 == y2, y1 + 1, y2)
    hr = y2 - y1
    wr = x2 - x1
    idx = jnp.arange(POOL)
    hs = y1[..., None] + (idx * hr[..., None]) // POOL
    he = y1[..., None] + ((idx + 1) * hr[..., None] + POOL - 1) // POOL
    ws = x1[..., None] + (idx * wr[..., None]) // POOL
    we = x1[..., None] + ((idx + 1) * wr[..., None] + POOL - 1) // POOL
    nh = he - hs
    lvl = 31 - jax.lax.clz(nh)
    rowa = hs
    rowb = he - jnp.left_shift(1, lvl)
    return lvl, rowa, rowb, ws, we, valid


def kernel(spatial_features, gt_boxes):
    bdim, cdim, hdim, wdim = spatial_features.shape
    ndim = gt_boxes.shape[1]

    cb = 8
    st = pl.pallas_call(
        functools.partial(_build_kernel, cb=cb, h=hdim, w=wdim),
        grid=(bdim, cdim // cb),
        in_specs=[pl.BlockSpec((1, cb, hdim, wdim),
                               lambda b, ci: (b, ci, 0, 0))],
        out_specs=pl.BlockSpec((NLEV, 1, hdim, cb, wdim),
                               lambda b, ci: (0, b, 0, ci, 0)),
        out_shape=jax.ShapeDtypeStruct((NLEV, bdim, hdim, cdim, wdim),
                                       jnp.bfloat16),
        compiler_params=pltpu.CompilerParams(
            dimension_semantics=("arbitrary", "arbitrary"),
            vmem_limit_bytes=100 * 1024 * 1024,
        ),
    )(spatial_features)

    lvl, rowa, rowb, ws, we, valid = _box_tables(gt_boxes, hdim, wdim)

    def mk_index(i, which):
        def idx(b, n, lvl_r, rowa_r, rowb_r, ws_r, we_r, valid_r):
            row_r = rowa_r if which == 0 else rowb_r
            return (lvl_r[b, n, i], b, row_r[b, n, i], 0, 0)
        return idx

    in_specs = []
    for i in range(POOL):
        for which in (0, 1):
            in_specs.append(pl.BlockSpec((1, 1, 1, cdim, wdim),
                                         mk_index(i, which)))

    grid_spec = pltpu.PrefetchScalarGridSpec(
        num_scalar_prefetch=6,
        grid=(bdim, ndim),
        in_specs=in_specs,
        out_specs=pl.BlockSpec((1, 1, POOL, cdim, POOL),
                               lambda b, n, *_: (b, n, 0, 0, 0)),
    )
    out = pl.pallas_call(
        functools.partial(_pool_kernel, c=cdim, w=wdim),
        grid_spec=grid_spec,
        out_shape=jax.ShapeDtypeStruct((bdim, ndim, POOL, cdim, POOL),
                                       jnp.float32),
        compiler_params=pltpu.CompilerParams(
            dimension_semantics=("arbitrary", "arbitrary"),
        ),
    )(lvl, rowa, rowb, ws, we, valid, *([st] * (2 * POOL)))
    return jnp.transpose(out, (0, 1, 3, 2, 4))


# speedup vs baseline: 18.3659x; 1.1750x over previous
import functools

import jax
import jax.numpy as jnp
from jax.experimental import pallas as pl
from jax.experimental.pallas import tpu as pltpu

POOL = 7
NLEV = 6
WWIN = 64


def _build_kernel(f_ref, st_ref, *, cb, h, w):
    cur = jnp.swapaxes(f_ref[0].astype(jnp.bfloat16), 0, 1)
    st_ref[0, 0] = cur
    for k in range(1, NLEV):
        s = 1 << (k - 1)
        shifted = jnp.concatenate(
            [cur[s:], jnp.broadcast_to(cur[h - 1:], (s, cb, w))],
            axis=0)
        cur = jnp.maximum(cur, shifted)
        st_ref[k, 0] = cur


def _pool_kernel(lvl_ref, rowa_ref, rowb_ref, ws_ref, we_ref, valid_ref,
                 *refs, c, w, nb):
    row_refs = refs[:2 * POOL * nb]
    o_ref = refs[2 * POOL * nb]
    b = pl.program_id(0)
    nid = pl.program_id(1)
    wids = jax.lax.broadcasted_iota(jnp.int32, (1, w), 1)
    neg = jnp.bfloat16(-jnp.inf)
    for s in range(nb):
        n = nid * nb + s
        masks = []
        for j in range(POOL):
            wsj = ws_ref[b, n, j]
            wej = we_ref[b, n, j]
            masks.append((wids >= wsj) & (wids < wej))
        vmask = valid_ref[b, n] != 0
        for i in range(POOL):
            ra = row_refs[2 * (s * POOL + i)][0, 0, 0]
            rb = row_refs[2 * (s * POOL + i) + 1][0, 0, 0]
            row2 = jnp.maximum(ra, rb)
            cols = [jnp.max(jnp.where(masks[j], row2, neg), axis=1)
                    for j in range(POOL)]
            vals = jnp.stack(cols, axis=1).astype(jnp.float32)
            vals = jnp.where(vmask, vals, jnp.float32(0.0))
            o_ref[0, s, i] = vals


def _box_tables(gt_boxes, hdim, wdim):
    boxes = gt_boxes.astype(jnp.int32)
    valid = jnp.any(boxes != 0, axis=2).astype(jnp.int32)
    x1 = jnp.clip(boxes[..., 0], 0, wdim - 1)
    y1 = jnp.clip(boxes[..., 1], 0, hdim - 1)
    x2 = jnp.clip(boxes[..., 2], 0, wdim - 1)
    y2 = jnp.clip(boxes[..., 3], 0, hdim - 1)
    x1 = jnp.minimum(x1, x2)
    x2 = jnp.maximum(x1, x2)
    y1 = jnp.minimum(y1, y2)
    y2 = jnp.maximum(y1, y2)
    x2 = jnp.where(x1 == x2, x1 + 1, x2)
    y2 = jnp.where(y1 == y2, y1 + 1, y2)
    hr = y2 - y1
    wr = x2 - x1
    idx = jnp.arange(POOL)
    hs = y1[..., None] + (idx * hr[..., None]) // POOL
    he = y1[..., None] + ((idx + 1) * hr[..., None] + POOL - 1) // POOL
    ws = x1[..., None] + (idx * wr[..., None]) // POOL
    we = x1[..., None] + ((idx + 1) * wr[..., None] + POOL - 1) // POOL
    nh = he - hs
    lvl = 31 - jax.lax.clz(nh)
    rowa = hs
    rowb = he - jnp.left_shift(1, lvl)
    return lvl, rowa, rowb, ws, we, valid


def kernel(spatial_features, gt_boxes):
    bdim, cdim, hdim, wdim = spatial_features.shape
    ndim = gt_boxes.shape[1]

    cb = 8
    st = pl.pallas_call(
        functools.partial(_build_kernel, cb=cb, h=hdim, w=wdim),
        grid=(bdim, cdim // cb),
        in_specs=[pl.BlockSpec((1, cb, hdim, wdim),
                               lambda b, ci: (b, ci, 0, 0))],
        out_specs=pl.BlockSpec((NLEV, 1, hdim, cb, wdim),
                               lambda b, ci: (0, b, 0, ci, 0)),
        out_shape=jax.ShapeDtypeStruct((NLEV, bdim, hdim, cdim, wdim),
                                       jnp.bfloat16),
        compiler_params=pltpu.CompilerParams(
            dimension_semantics=("arbitrary", "arbitrary"),
            vmem_limit_bytes=100 * 1024 * 1024,
        ),
    )(spatial_features)

    lvl, rowa, rowb, ws, we, valid = _box_tables(gt_boxes, hdim, wdim)

    nb = 2

    def mk_index(s, i, which):
        def idx(b, nid, lvl_r, rowa_r, rowb_r, ws_r, we_r, valid_r):
            row_r = rowa_r if which == 0 else rowb_r
            n = nid * nb + s
            return (lvl_r[b, n, i], b, row_r[b, n, i], 0, 0)
        return idx

    in_specs = []
    for s in range(nb):
        for i in range(POOL):
            for which in (0, 1):
                in_specs.append(pl.BlockSpec((1, 1, 1, cdim, wdim),
                                             mk_index(s, i, which)))

    grid_spec = pltpu.PrefetchScalarGridSpec(
        num_scalar_prefetch=6,
        grid=(bdim, ndim // nb),
        in_specs=in_specs,
        out_specs=pl.BlockSpec((1, nb, POOL, cdim, POOL),
                               lambda b, nid, *_: (b, nid, 0, 0, 0)),
    )
    out = pl.pallas_call(
        functools.partial(_pool_kernel, c=cdim, w=wdim, nb=nb),
        grid_spec=grid_spec,
        out_shape=jax.ShapeDtypeStruct((bdim, ndim, POOL, cdim, POOL),
                                       jnp.float32),
        compiler_params=pltpu.CompilerParams(
            dimension_semantics=("arbitrary", "arbitrary"),
        ),
    )(lvl, rowa, rowb, ws, we, valid, *([st] * (2 * POOL * nb)))
    return jnp.transpose(out, (0, 1, 3, 2, 4))


# speedup vs baseline: 19.9107x; 1.0841x over previous
import functools

import jax
import jax.numpy as jnp
from jax.experimental import pallas as pl
from jax.experimental.pallas import tpu as pltpu

POOL = 7
NLEV = 6
WWIN = 64


def _build_kernel(f_ref, st_ref, *, cb, h, w):
    cur = jnp.swapaxes(f_ref[0].astype(jnp.bfloat16), 0, 1)
    st_ref[0, 0] = cur
    for k in range(1, NLEV):
        s = 1 << (k - 1)
        shifted = jnp.concatenate(
            [cur[s:], jnp.broadcast_to(cur[h - 1:], (s, cb, w))],
            axis=0)
        cur = jnp.maximum(cur, shifted)
        st_ref[k, 0] = cur


def _pool_kernel(lvl_ref, rowa_ref, rowb_ref, ws_ref, we_ref, valid_ref,
                 *refs, c, w, nb):
    row_refs = refs[:2 * POOL * nb]
    o_ref = refs[2 * POOL * nb]
    b = pl.program_id(0)
    nid = pl.program_id(1)
    wids = jax.lax.broadcasted_iota(jnp.int32, (1, w), 1)
    neg = jnp.bfloat16(-jnp.inf)
    for s in range(nb):
        n = nid * nb + s
        masks = []
        for j in range(POOL):
            wsj = ws_ref[b, n, j]
            wej = we_ref[b, n, j]
            masks.append((wids >= wsj) & (wids < wej))
        vmask = valid_ref[b, n] != 0
        for i in range(POOL):
            ra = row_refs[2 * (s * POOL + i)][0, 0, 0]
            rb = row_refs[2 * (s * POOL + i) + 1][0, 0, 0]
            row2 = jnp.maximum(ra, rb)
            cols = [jnp.max(jnp.where(masks[j], row2, neg), axis=1)
                    for j in range(POOL)]
            vals = jnp.stack(cols, axis=1).astype(jnp.float32)
            vals = jnp.where(vmask, vals, jnp.float32(0.0))
            o_ref[0, s, i] = vals


def _box_tables(gt_boxes, hdim, wdim):
    boxes = gt_boxes.astype(jnp.int32)
    valid = jnp.any(boxes != 0, axis=2).astype(jnp.int32)
    x1 = jnp.clip(boxes[..., 0], 0, wdim - 1)
    y1 = jnp.clip(boxes[..., 1], 0, hdim - 1)
    x2 = jnp.clip(boxes[..., 2], 0, wdim - 1)
    y2 = jnp.clip(boxes[..., 3], 0, hdim - 1)
    x1 = jnp.minimum(x1, x2)
    x2 = jnp.maximum(x1, x2)
    y1 = jnp.minimum(y1, y2)
    y2 = jnp.maximum(y1, y2)
    x2 = jnp.where(x1 == x2, x1 + 1, x2)
    y2 = jnp.where(y1 == y2, y1 + 1, y2)
    hr = y2 - y1
    wr = x2 - x1
    idx = jnp.arange(POOL)
    hs = y1[..., None] + (idx * hr[..., None]) // POOL
    he = y1[..., None] + ((idx + 1) * hr[..., None] + POOL - 1) // POOL
    ws = x1[..., None] + (idx * wr[..., None]) // POOL
    we = x1[..., None] + ((idx + 1) * wr[..., None] + POOL - 1) // POOL
    nh = he - hs
    lvl = 31 - jax.lax.clz(nh)
    rowa = hs
    rowb = he - jnp.left_shift(1, lvl)
    return lvl, rowa, rowb, ws, we, valid


def kernel(spatial_features, gt_boxes):
    bdim, cdim, hdim, wdim = spatial_features.shape
    ndim = gt_boxes.shape[1]

    cb = 8
    st = pl.pallas_call(
        functools.partial(_build_kernel, cb=cb, h=hdim, w=wdim),
        grid=(bdim, cdim // cb),
        in_specs=[pl.BlockSpec((1, cb, hdim, wdim),
                               lambda b, ci: (b, ci, 0, 0))],
        out_specs=pl.BlockSpec((NLEV, 1, hdim, cb, wdim),
                               lambda b, ci: (0, b, 0, ci, 0)),
        out_shape=jax.ShapeDtypeStruct((NLEV, bdim, hdim, cdim, wdim),
                                       jnp.bfloat16),
        compiler_params=pltpu.CompilerParams(
            dimension_semantics=("arbitrary", "arbitrary"),
            vmem_limit_bytes=100 * 1024 * 1024,
        ),
    )(spatial_features)

    lvl, rowa, rowb, ws, we, valid = _box_tables(gt_boxes, hdim, wdim)

    nb = 4

    def mk_index(s, i, which):
        def idx(b, nid, lvl_r, rowa_r, rowb_r, ws_r, we_r, valid_r):
            row_r = rowa_r if which == 0 else rowb_r
            n = nid * nb + s
            return (lvl_r[b, n, i], b, row_r[b, n, i], 0, 0)
        return idx

    in_specs = []
    for s in range(nb):
        for i in range(POOL):
            for which in (0, 1):
                in_specs.append(pl.BlockSpec((1, 1, 1, cdim, wdim),
                                             mk_index(s, i, which)))

    grid_spec = pltpu.PrefetchScalarGridSpec(
        num_scalar_prefetch=6,
        grid=(bdim, ndim // nb),
        in_specs=in_specs,
        out_specs=pl.BlockSpec((1, nb, POOL, cdim, POOL),
                               lambda b, nid, *_: (b, nid, 0, 0, 0)),
    )
    out = pl.pallas_call(
        functools.partial(_pool_kernel, c=cdim, w=wdim, nb=nb),
        grid_spec=grid_spec,
        out_shape=jax.ShapeDtypeStruct((bdim, ndim, POOL, cdim, POOL),
                                       jnp.float32),
        compiler_params=pltpu.CompilerParams(
            dimension_semantics=("arbitrary", "arbitrary"),
        ),
    )(lvl, rowa, rowb, ws, we, valid, *([st] * (2 * POOL * nb)))
    return jnp.transpose(out, (0, 1, 3, 2, 4))
